# baseline (speedup 1.0000x reference)
import functools

import jax
import jax.numpy as jnp
from jax import lax
from jax.experimental import pallas as pl
from jax.experimental.pallas import tpu as pltpu
from jax.experimental.pallas import tpu_sc as plsc

N = 4096
D = 128
E = 131072
DE = 16
NN = N * N

NC, NS = 2, 16
NW = NC * NS
EPW = E // NW
COLS = 128
ROWS = EPW // COLS
GRP = COLS // 16

BM = 1024
EB = E // (N // BM)


def _tc_body(zi_ref, zall_ref, eat_ref, out_ref, en_ref):
    zz = lax.dot_general(
        zi_ref[...], zall_ref[...],
        (((1,), (1,)), ((), ())),
        preferred_element_type=jnp.float32,
    )
    out_ref[...] = 1.0 / (1.0 + jnp.exp(-zz))
    en_ref[...] = jnp.exp(jnp.sum(eat_ref[...], axis=0) * (-1.0 / DE))


def _dense_tiled(z, eat):
    return pl.pallas_call(
        _tc_body,
        grid=(N // BM,),
        in_specs=[
            pl.BlockSpec((BM, D), lambda i: (i, 0)),
            pl.BlockSpec((N, D), lambda i: (0, 0)),
            pl.BlockSpec((DE, EB), lambda i: (0, i)),
        ],
        out_specs=[
            pl.BlockSpec((BM, N), lambda i: (i, 0)),
            pl.BlockSpec((EB,), lambda i: (i,)),
        ],
        out_shape=[
            jax.ShapeDtypeStruct((N, N), jnp.float32),
            jax.ShapeDtypeStruct((E,), jnp.float32),
        ],
    )(z, z, eat)


def _tiled_words(dense):
    return dense.reshape(N // 8, 8, N // 128, 128).transpose(0, 2, 1, 3).reshape(NN)


def _untiled(flat):
    return flat.reshape(N // 8, N // 128, 8, 128).transpose(0, 2, 1, 3).reshape(N, N)


_mesh = plsc.VectorSubcoreMesh(
    core_axis_name="c", subcore_axis_name="s", num_cores=NC, num_subcores=NS)


NQ = 4
CH = EPW // NQ


@functools.partial(
    pl.kernel,
    mesh=_mesh,
    compiler_params=pltpu.CompilerParams(needs_layout_passes=False),
    scratch_types=[
        pltpu.VMEM((EPW,), jnp.int32),
        pltpu.VMEM((EPW,), jnp.int32),
        pltpu.VMEM((EPW,), jnp.float32),
    ] + [pltpu.VMEM((CH,), jnp.int32) for _ in range(NQ)]
      + [pltpu.VMEM((CH,), jnp.float32) for _ in range(NQ)]
      + [pltpu.SemaphoreType.DMA for _ in range(NQ)]
      + [
        pltpu.SemaphoreType.DMA,
        pltpu.SemaphoreType.DMA,
    ],
)
def _sc_fix(out_hbm, ei_hbm, en_hbm, r_v, c_v, en_v,
            i0, i1, i2, i3, y0, y1, y2, y3, g0, g1, g2, g3, lsem, ssem):
    idxs = [i0, i1, i2, i3]
    ys = [y0, y1, y2, y3]
    gsems = [g0, g1, g2, g3]
    wid = lax.axis_index("s") * NC + lax.axis_index("c")
    base = wid * EPW
    pltpu.sync_copy(ei_hbm.at[0, pl.ds(base, EPW)], r_v)
    pltpu.sync_copy(ei_hbm.at[1, pl.ds(base, EPW)], c_v)
    en_load = pltpu.async_copy(en_hbm.at[pl.ds(base, EPW)], en_v, lsem)

    gathers = []
    for q in range(NQ):
        def build(j, _, _q=q):
            for k in range(GRP):
                off = _q * CH + j * COLS + k * 16
                r = r_v[pl.ds(off, 16)]
                c = c_v[pl.ds(off, 16)]
                idxs[_q][pl.ds(j * COLS + k * 16, 16)] = (
                    ((r >> 3) << 15) | ((c >> 7) << 10)
                    | ((r & 7) << 7) | (c & 127)
                )
            return 0

        lax.fori_loop(0, CH // COLS, build, 0)
        gathers.append(pltpu.async_copy(out_hbm.at[idxs[q]], ys[q], gsems[q]))

    en_load.wait()
    scatters = []
    for q in range(NQ):
        gathers[q].wait()

        def fix(j, _, _q=q):
            y = ys[_q][pl.ds(j * 16, 16)]
            en = en_v[pl.ds(_q * CH + j * 16, 16)]
            ys[_q][pl.ds(j * 16, 16)] = y / (y + en * (1.0 - y))
            return 0

        lax.fori_loop(0, CH // 16, fix, 0)
        scatters.append(pltpu.async_copy(ys[q], out_hbm.at[idxs[q]], ssem))
    for sc in scatters:
        sc.wait()


def kernel(z, edge_index, edge_attr):
    dense, en = _dense_tiled(z, edge_attr.T)
    ref = jax.new_ref(_tiled_words(dense))
    _sc_fix(ref, edge_index, en)
    return _untiled(ref[...])


# speedup vs baseline: 1.0049x; 1.0049x over previous
import functools

import jax
import jax.numpy as jnp
from jax import lax
from jax.experimental import pallas as pl
from jax.experimental.pallas import tpu as pltpu
from jax.experimental.pallas import tpu_sc as plsc

N = 4096
D = 128
E = 131072
DE = 16
NN = N * N

NC, NS = 2, 16
NW = NC * NS
EPW = E // NW
COLS = 128
ROWS = EPW // COLS
GRP = COLS // 16

BM = 512
EB = E // (N // BM)


def _tc_body(zi_ref, zall_ref, eat_ref, out_ref, en_ref):
    zz = lax.dot_general(
        zi_ref[...], zall_ref[...],
        (((1,), (1,)), ((), ())),
        preferred_element_type=jnp.float32,
    )
    out_ref[...] = 1.0 / (1.0 + jnp.exp(-zz))
    en_ref[...] = jnp.exp(jnp.sum(eat_ref[...], axis=0) * (-1.0 / DE))


def _dense_tiled(z, eat):
    return pl.pallas_call(
        _tc_body,
        grid=(N // BM,),
        in_specs=[
            pl.BlockSpec((BM, D), lambda i: (i, 0)),
            pl.BlockSpec((N, D), lambda i: (0, 0)),
            pl.BlockSpec((DE, EB), lambda i: (0, i)),
        ],
        out_specs=[
            pl.BlockSpec((BM, N), lambda i: (i, 0)),
            pl.BlockSpec((EB,), lambda i: (i,)),
        ],
        out_shape=[
            jax.ShapeDtypeStruct((N, N), jnp.float32),
            jax.ShapeDtypeStruct((E,), jnp.float32),
        ],
    )(z, z, eat)


def _tiled_words(dense):
    return dense.reshape(N // 8, 8, N // 128, 128).transpose(0, 2, 1, 3).reshape(NN)


def _untiled(flat):
    return flat.reshape(N // 8, N // 128, 8, 128).transpose(0, 2, 1, 3).reshape(N, N)


_mesh = plsc.VectorSubcoreMesh(
    core_axis_name="c", subcore_axis_name="s", num_cores=NC, num_subcores=NS)


NQ = 8
CH = EPW // NQ


@functools.partial(
    pl.kernel,
    mesh=_mesh,
    compiler_params=pltpu.CompilerParams(needs_layout_passes=False),
    scratch_types=[
        pltpu.VMEM((EPW,), jnp.int32),
        pltpu.VMEM((EPW,), jnp.int32),
        pltpu.VMEM((EPW,), jnp.float32),
    ] + [pltpu.VMEM((CH,), jnp.int32) for _ in range(NQ)]
      + [pltpu.VMEM((CH,), jnp.float32) for _ in range(NQ)]
      + [pltpu.SemaphoreType.DMA for _ in range(NQ)]
      + [
        pltpu.SemaphoreType.DMA,
        pltpu.SemaphoreType.DMA,
    ],
)
def _sc_fix(out_hbm, ei_hbm, en_hbm, r_v, c_v, en_v,
            i0, i1, i2, i3, i4, i5, i6, i7,
            y0, y1, y2, y3, y4, y5, y6, y7,
            g0, g1, g2, g3, g4, g5, g6, g7, lsem, ssem):
    idxs = [i0, i1, i2, i3, i4, i5, i6, i7]
    ys = [y0, y1, y2, y3, y4, y5, y6, y7]
    gsems = [g0, g1, g2, g3, g4, g5, g6, g7]
    wid = lax.axis_index("s") * NC + lax.axis_index("c")
    base = wid * EPW
    pltpu.sync_copy(ei_hbm.at[0, pl.ds(base, EPW)], r_v)
    pltpu.sync_copy(ei_hbm.at[1, pl.ds(base, EPW)], c_v)
    en_load = pltpu.async_copy(en_hbm.at[pl.ds(base, EPW)], en_v, lsem)

    gathers = []
    for q in range(NQ):
        def build(j, _, _q=q):
            for k in range(GRP):
                off = _q * CH + j * COLS + k * 16
                r = r_v[pl.ds(off, 16)]
                c = c_v[pl.ds(off, 16)]
                idxs[_q][pl.ds(j * COLS + k * 16, 16)] = (
                    ((r >> 3) << 15) | ((c >> 7) << 10)
                    | ((r & 7) << 7) | (c & 127)
                )
            return 0

        lax.fori_loop(0, CH // COLS, build, 0)
        gathers.append(pltpu.async_copy(out_hbm.at[idxs[q]], ys[q], gsems[q]))

    en_load.wait()
    scatters = []
    for q in range(NQ):
        gathers[q].wait()

        def fix(j, _, _q=q):
            y = ys[_q][pl.ds(j * 16, 16)]
            en = en_v[pl.ds(_q * CH + j * 16, 16)]
            ys[_q][pl.ds(j * 16, 16)] = y / (y + en * (1.0 - y))
            return 0

        lax.fori_loop(0, CH // 16, fix, 0)
        scatters.append(pltpu.async_copy(ys[q], out_hbm.at[idxs[q]], ssem))
    for sc in scatters:
        sc.wait()


def kernel(z, edge_index, edge_attr):
    dense, en = _dense_tiled(z, edge_attr.T)
    ref = jax.new_ref(_tiled_words(dense))
    _sc_fix(ref, edge_index, en)
    return _untiled(ref[...])
